# baseline (speedup 1.0000x reference)
import functools

import jax
import jax.numpy as jnp
from jax import lax
from jax.experimental import pallas as pl
from jax.experimental.pallas import tpu as pltpu
from jax.experimental.pallas import tpu_sc as plsc

_DIM = 128
_E = 320000
_V0, _V1, _V2 = 5, 6, 2
_NCOMB = _V0 * _V1 * _V2
_NC, _NS = 2, 16
_NW = _NC * _NS
_CHUNK = 256
_NCHUNK = _E // _CHUNK
_NBUF = 3
_KMAX = -(-(-(-_NCHUNK // _NW)) // _NBUF) * _NBUF
_G16 = _CHUNK // 16

_mesh = plsc.VectorSubcoreMesh(core_axis_name="c", subcore_axis_name="s")


@functools.partial(
    pl.kernel,
    mesh=_mesh,
    out_type=jax.ShapeDtypeStruct((_E, _DIM), jnp.float32),
    scratch_types=[
        pltpu.VMEM((_V0, _DIM), jnp.float32),
        pltpu.VMEM((_V1, _DIM), jnp.float32),
        pltpu.VMEM((_V2, _DIM), jnp.float32),
        pltpu.VMEM((_NCOMB * _DIM,), jnp.float32),
        pltpu.VMEM((_NBUF, 3, _CHUNK), jnp.int32),
        pltpu.VMEM((_CHUNK + 16,), jnp.int32),
        pltpu.VMEM((_NBUF, _CHUNK, _DIM), jnp.float32),
        pltpu.SemaphoreType.DMA,
        pltpu.SemaphoreType.DMA,
        pltpu.SemaphoreType.DMA,
        pltpu.SemaphoreType.DMA,
        pltpu.SemaphoreType.DMA,
        pltpu.SemaphoreType.DMA,
    ],
)
def _sc_encode(ef_hbm, w0_hbm, w1_hbm, w2_hbm, out_hbm,
               w0_v, w1_v, w2_v, tab_v, ef_v, cb_v, out_v,
               sem_ef0, sem_ef1, sem_ef2, sem_out0, sem_out1, sem_out2):
    wid = lax.axis_index("s") * _NC + lax.axis_index("c")
    sem_ef = (sem_ef0, sem_ef1, sem_ef2)
    sem_out = (sem_out0, sem_out1, sem_out2)

    def ef_slice(c):
        off = pl.multiple_of(c * _CHUNK, 128)
        return ef_hbm.at[:, pl.ds(off, _CHUNK)]

    def out_slice(c):
        off = pl.multiple_of(c * _CHUNK, 128)
        return out_hbm.at[pl.ds(off, _CHUNK)]

    pltpu.async_copy(ef_slice(wid), ef_v.at[0], sem_ef[0])

    pltpu.sync_copy(w0_hbm, w0_v)
    pltpu.sync_copy(w1_hbm, w1_v)
    pltpu.sync_copy(w2_hbm, w2_v)

    r0 = [[w0_v[i, pl.ds(j * 16, 16)] for j in range(8)] for i in range(_V0)]
    r1 = [[w1_v[i, pl.ds(j * 16, 16)] for j in range(8)] for i in range(_V1)]
    r2 = [[w2_v[i, pl.ds(j * 16, 16)] for j in range(8)] for i in range(_V2)]
    for c0 in range(_V0):
        for c1 in range(_V1):
            t01 = [r0[c0][j] + r1[c1][j] for j in range(8)]
            for c2 in range(_V2):
                base = ((c0 * _V1 + c1) * _V2 + c2) * _DIM
                for j in range(8):
                    tab_v[pl.ds(base + j * 16, 16)] = t01[j] + r2[c2][j]

    def ring_body(kr, carry):
        for b in range(_NBUF):
            k = kr * _NBUF + b
            eb = b
            en = (b + 1) % _NBUF
            c = k * _NW + wid
            valid = c < _NCHUNK
            c_next = c + _NW

            @pl.when(valid)
            def _wait_ef(eb=eb, c=c):
                pltpu.make_async_copy(ef_slice(c), ef_v.at[eb], sem_ef[eb]).wait()

            @pl.when(c_next < _NCHUNK)
            def _issue_ef(en=en, c_next=c_next):
                pltpu.async_copy(ef_slice(c_next), ef_v.at[en], sem_ef[en])

            @pl.when(kr >= 1)
            def _wait_out(b=b, c=c):
                pltpu.make_async_copy(
                    out_v.at[b], out_slice(c - _NBUF * _NW), sem_out[b]).wait()

            @pl.when(valid)
            def _compute(b=b, eb=eb, c=c):
                for g in range(_G16):
                    e0 = ef_v[eb, 0, pl.ds(g * 16, 16)]
                    e1 = ef_v[eb, 1, pl.ds(g * 16, 16)]
                    e2 = ef_v[eb, 2, pl.ds(g * 16, 16)]
                    comb = (e0 * (_V1 * _V2) + e1 * _V2 + e2) * _DIM
                    cb_v[pl.ds(g * 16, 16)] = comb

                def edge_body(e16, cc):
                    e = e16 * 16
                    bv = cb_v[pl.ds(e, 16)]
                    for u in range(16):
                        bb = bv[u]
                        row = [tab_v[pl.ds(bb + j * 16, 16)] for j in range(8)]
                        for j in range(8):
                            out_v[b, e + u, pl.ds(j * 16, 16)] = row[j]
                    return cc

                lax.fori_loop(0, _CHUNK // 16, edge_body, 0)

                pltpu.async_copy(out_v.at[b], out_slice(c), sem_out[b])

        return carry

    lax.fori_loop(0, _KMAX // _NBUF, ring_body, 0)

    for b in range(_NBUF):
        k_last = _KMAX - _NBUF + b
        c_last = k_last * _NW + wid

        @pl.when(c_last < _NCHUNK)
        def _drain(b=b, c_last=c_last):
            pltpu.make_async_copy(out_v.at[b], out_slice(c_last), sem_out[b]).wait()


def kernel(edge_feat, W0, W1, W2):
    return _sc_encode(edge_feat.T, W0, W1, W2)


# speedup vs baseline: 1.0175x; 1.0175x over previous
import functools

import jax
import jax.numpy as jnp
from jax import lax
from jax.experimental import pallas as pl
from jax.experimental.pallas import tpu as pltpu
from jax.experimental.pallas import tpu_sc as plsc

_DIM = 128
_E = 320000
_V0, _V1, _V2 = 5, 6, 2
_NCOMB = _V0 * _V1 * _V2
_NC, _NS = 2, 16
_NW = _NC * _NS
_NBLK = _E // 128
_CHUNK = 384
_KMAX = 26
_G16 = _CHUNK // 16
_TAIL = 128

_mesh = plsc.VectorSubcoreMesh(core_axis_name="c", subcore_axis_name="s")


@functools.partial(
    pl.kernel,
    mesh=_mesh,
    out_type=jax.ShapeDtypeStruct((_E, _DIM), jnp.float32),
    scratch_types=[
        pltpu.VMEM((_V0, _DIM), jnp.float32),
        pltpu.VMEM((_V1, _DIM), jnp.float32),
        pltpu.VMEM((_V2, _DIM), jnp.float32),
        pltpu.VMEM((_NCOMB * _DIM,), jnp.float32),
        pltpu.VMEM((2, 3, _CHUNK), jnp.int32),
        pltpu.VMEM((_CHUNK + 16,), jnp.int32),
        pltpu.VMEM((2, _CHUNK, _DIM), jnp.float32),
        pltpu.SemaphoreType.DMA,
        pltpu.SemaphoreType.DMA,
        pltpu.SemaphoreType.DMA,
        pltpu.SemaphoreType.DMA,
    ],
)
def _sc_encode(ef_hbm, w0_hbm, w1_hbm, w2_hbm, out_hbm,
               w0_v, w1_v, w2_v, tab_v, ef_v, cb_v, out_v,
               sem_ef0, sem_ef1, sem_out0, sem_out1):
    wid = lax.axis_index("s") * _NC + lax.axis_index("c")
    sem_ef = (sem_ef0, sem_ef1)
    sem_out = (sem_out0, sem_out1)

    row0 = lax.shift_right_logical(wid * (_NBLK // 4), 3) * 128
    has_tail = lax.bitwise_and(wid, 7) == 7

    def ef_slice(off, n):
        return ef_hbm.at[:, pl.ds(pl.multiple_of(off, 128), n)]

    def out_slice(off, n):
        return out_hbm.at[pl.ds(pl.multiple_of(off, 128), n)]

    pltpu.async_copy(ef_slice(row0, _CHUNK), ef_v.at[0], sem_ef[0])

    pltpu.sync_copy(w0_hbm, w0_v)
    pltpu.sync_copy(w1_hbm, w1_v)
    pltpu.sync_copy(w2_hbm, w2_v)

    r0 = [[w0_v[i, pl.ds(j * 16, 16)] for j in range(8)] for i in range(_V0)]
    r1 = [[w1_v[i, pl.ds(j * 16, 16)] for j in range(8)] for i in range(_V1)]
    r2 = [[w2_v[i, pl.ds(j * 16, 16)] for j in range(8)] for i in range(_V2)]
    for c0 in range(_V0):
        for c1 in range(_V1):
            t01 = [r0[c0][j] + r1[c1][j] for j in range(8)]
            for c2 in range(_V2):
                base = ((c0 * _V1 + c1) * _V2 + c2) * _DIM
                for j in range(8):
                    tab_v[pl.ds(base + j * 16, 16)] = t01[j] + r2[c2][j]

    def make_offsets(eb, ngroups):
        for g in range(ngroups):
            e0 = ef_v[eb, 0, pl.ds(g * 16, 16)]
            e1 = ef_v[eb, 1, pl.ds(g * 16, 16)]
            e2 = ef_v[eb, 2, pl.ds(g * 16, 16)]
            comb = (e0 * (_V1 * _V2) + e1 * _V2 + e2) * _DIM
            cb_v[pl.ds(g * 16, 16)] = comb

    def copy_rows(ob, ngroups):
        def edge_body(e16, cc):
            e = e16 * 16
            bv = cb_v[pl.ds(e, 16)]
            for u in range(16):
                bb = bv[u]
                row = [tab_v[pl.ds(bb + j * 16, 16)] for j in range(8)]
                for j in range(8):
                    out_v[ob, e + u, pl.ds(j * 16, 16)] = row[j]
            return cc

        lax.fori_loop(0, ngroups, edge_body, 0)

    def pair_body(k2, carry):
        for b in range(2):
            k = k2 * 2 + b
            off = row0 + k * _CHUNK

            pltpu.make_async_copy(
                ef_slice(off, _CHUNK), ef_v.at[b], sem_ef[b]).wait()

            @pl.when(k < _KMAX - 1)
            def _issue_ef(b=b, off=off):
                pltpu.async_copy(
                    ef_slice(off + _CHUNK, _CHUNK), ef_v.at[1 - b],
                    sem_ef[1 - b])

            @pl.when(k2 >= 1)
            def _wait_out(b=b, off=off):
                pltpu.make_async_copy(
                    out_v.at[b], out_slice(off - 2 * _CHUNK, _CHUNK),
                    sem_out[b]).wait()

            make_offsets(b, _G16)
            copy_rows(b, _CHUNK // 16)
            pltpu.async_copy(out_v.at[b], out_slice(off, _CHUNK), sem_out[b])

        return carry

    lax.fori_loop(0, _KMAX // 2, pair_body, 0)

    end0 = row0 + (_KMAX - 2) * _CHUNK
    pltpu.make_async_copy(out_v.at[0], out_slice(end0, _CHUNK), sem_out[0]).wait()

    @pl.when(has_tail)
    def _tail():
        toff = row0 + _KMAX * _CHUNK
        pltpu.sync_copy(ef_slice(toff, _TAIL), ef_v.at[0, :, pl.ds(0, _TAIL)])
        make_offsets(0, _TAIL // 16)
        copy_rows(0, _TAIL // 16)
        pltpu.sync_copy(out_v.at[0, pl.ds(0, _TAIL)], out_slice(toff, _TAIL))

    end1 = row0 + (_KMAX - 1) * _CHUNK
    pltpu.make_async_copy(out_v.at[1], out_slice(end1, _CHUNK), sem_out[1]).wait()


def kernel(edge_feat, W0, W1, W2):
    return _sc_encode(edge_feat.T, W0, W1, W2)
